# baseline (speedup 1.0000x reference)
import jax
import jax.numpy as jnp
from jax import lax
from jax.experimental import pallas as pl
from jax.experimental.pallas import tpu as pltpu
from jax.experimental.pallas import tpu_sc as plsc

N = 100000
IN_CH = 384
DIM = 128
M = 4096
BW2 = 4.0
ITERS = 4

NC = 2
NS = 16
NW = NC * NS
B = 128
CH = 25
ROWS_W = B * CH
N2 = NW * ROWS_W

BI = 256
NBLK = M // BI


def _segsum_body(emb_hbm, idx_hbm, zeros_hbm, out_hbm, idx_v, emb_v, acc_sh):
    c = lax.axis_index("c")
    s = lax.axis_index("s")
    w = c * NS + s

    @pl.when(s == 0)
    def _():
        pltpu.sync_copy(zeros_hbm, acc_sh)

    for comp in range(4):
        pltpu.sync_copy(emb_hbm.at[pl.ds(comp * N2 + w * ROWS_W, ROWS_W)],
                        emb_v.at[pl.ds(comp * ROWS_W, ROWS_W)])
    pltpu.sync_copy(idx_hbm.at[w], idx_v)
    plsc.subcore_barrier()

    def body(j, carry):
        comp = j // CH
        pltpu.sync_copy(emb_v.at[pl.ds(comp * ROWS_W + (j % CH) * B, B)],
                        acc_sh.at[idx_v.at[j]], add=True)
        return carry

    lax.fori_loop(0, 4 * CH, body, 0)
    plsc.subcore_barrier()

    @pl.when(s == 0)
    def _():
        pltpu.sync_copy(acc_sh, out_hbm.at[c])


_segsum_call = pl.kernel(
    _segsum_body,
    out_type=jax.ShapeDtypeStruct((NC, 4 * M), jnp.float32),
    mesh=plsc.VectorSubcoreMesh(core_axis_name="c", subcore_axis_name="s"),
    scratch_types=[
        pltpu.VMEM((4 * CH, B), jnp.int32),
        pltpu.VMEM((4 * ROWS_W,), jnp.float32),
        pltpu.VMEM_SHARED((4 * M,), jnp.float32),
    ],
)


def _shift_body(parts_ref, out_ref, k_ref, col_ref, yt_ref, yt2_ref, yhl_ref):
    rows = []
    for comp in range(4):
        rows.append(parts_ref[0:1, pl.ds(comp * M, M)]
                    + parts_ref[1:2, pl.ds(comp * M, M)])
    cntr = jnp.maximum(rows[3], 1.0)
    xr = rows[0] / cntr
    yr = rows[1] / cntr
    zr = rows[2] / cntr
    sqr = xr * xr + yr * yr + zr * zr
    onesr = jnp.ones((1, M), jnp.float32)
    zerosr = jnp.zeros((4, M), jnp.float32)
    yt_ref[...] = jnp.concatenate([xr, yr, zr, onesr, zerosr], axis=0)
    col_ref[...] = jnp.transpose(
        jnp.concatenate([xr, yr, zr, sqr], axis=0))

    def kb(i, carry):
        r = pl.ds(i * BI, BI)
        xi = col_ref[r, 0:1]
        yi = col_ref[r, 1:2]
        zi = col_ref[r, 2:3]
        sqi = col_ref[r, 3:4]
        d2 = (sqi + sqr) - 2.0 * (xi * xr + yi * yr + zi * zr)
        k_ref[r, :] = (d2 <= BW2).astype(jnp.bfloat16)
        return carry

    lax.fori_loop(0, NBLK, kb, 0)

    def it(t, carry):
        yt = yt_ref[...]
        yth = yt.astype(jnp.bfloat16)
        ytl = (yt - yth.astype(jnp.float32)).astype(jnp.bfloat16)
        yhl_ref[...] = jnp.concatenate([yth, ytl], axis=0)

        def mm(i, c2):
            cslice = pl.ds(i * BI, BI)
            kcol = k_ref[:, cslice]
            z = jnp.dot(yhl_ref[...], kcol,
                        preferred_element_type=jnp.float32)
            zs = z[0:8, :] + z[8:16, :]
            yt2_ref[:, cslice] = zs / zs[3:4, :]
            return c2

        lax.fori_loop(0, NBLK, mm, 0)
        yt_ref[...] = yt2_ref[...]
        return carry

    lax.fori_loop(0, ITERS, it, 0)
    for comp in range(3):
        out_ref[pl.ds(comp * M, M)] = jnp.reshape(yt_ref[comp:comp + 1, :], (M,))
    out_ref[pl.ds(3 * M, M)] = jnp.reshape(yt_ref[3:4, :], (M,))


def _shift_call(parts2):
    return pl.pallas_call(
        _shift_body,
        out_shape=jax.ShapeDtypeStruct((4 * M,), jnp.float32),
        scratch_shapes=[
            pltpu.VMEM((M, M), jnp.bfloat16),
            pltpu.VMEM((M, 4), jnp.float32),
            pltpu.VMEM((8, M), jnp.float32),
            pltpu.VMEM((8, M), jnp.float32),
            pltpu.VMEM((16, M), jnp.bfloat16),
        ],
    )(parts2)


def _gather_body(tab_hbm, idx_hbm, out_hbm, tab_v, idx_v, out_v):
    c = lax.axis_index("c")
    s = lax.axis_index("s")
    w = c * NS + s
    pltpu.sync_copy(tab_hbm, tab_v)
    pltpu.sync_copy(idx_hbm.at[w], idx_v)

    def body(j, carry):
        def inner(k2, c2):
            idx16 = idx_v[j, pl.ds(k2 * 16, 16)]
            vals = plsc.load_gather(tab_v, [idx16])
            out_v[pl.ds(j * B + k2 * 16, 16)] = vals
            return c2

        lax.fori_loop(0, B // 16, inner, 0)
        return carry

    lax.fori_loop(0, 3 * CH, body, 0)
    for comp in range(3):
        pltpu.sync_copy(out_v.at[pl.ds(comp * ROWS_W, ROWS_W)],
                        out_hbm.at[pl.ds(comp * N2 + w * ROWS_W, ROWS_W)])


_gather_call = pl.kernel(
    _gather_body,
    out_type=jax.ShapeDtypeStruct((3 * N2,), jnp.float32),
    mesh=plsc.VectorSubcoreMesh(core_axis_name="c", subcore_axis_name="s"),
    compiler_params=pltpu.CompilerParams(needs_layout_passes=False),
    scratch_types=[
        pltpu.VMEM((4 * M,), jnp.float32),
        pltpu.VMEM((4 * CH, B), jnp.int32),
        pltpu.VMEM((3 * ROWS_W,), jnp.float32),
    ],
)


BN = 4096


def _mm_body(x_ref, wt_ref, o_ref):
    o_ref[...] = lax.dot_general(wt_ref[...], x_ref[...],
                                 (((1,), (1,)), ((), ())),
                                 preferred_element_type=jnp.float32)


def _mm_call(ins_fea, wt):
    return pl.pallas_call(
        _mm_body,
        grid=((N + BN - 1) // BN,),
        in_specs=[
            pl.BlockSpec((BN, IN_CH), lambda i: (i, 0)),
            pl.BlockSpec((DIM, IN_CH), lambda i: (0, 0)),
        ],
        out_specs=pl.BlockSpec((DIM, BN), lambda i: (0, i)),
        out_shape=jax.ShapeDtypeStruct((DIM + 3, N), jnp.float32),
    )(ins_fea, wt)


def _cw_body(o_in_ref, c_ref, o_ref):
    del o_in_ref
    o_ref[0:3, :] = c_ref[...]


def _cw_call(feats_out, cent):
    return pl.pallas_call(
        _cw_body,
        grid=((N + BN - 1) // BN,),
        in_specs=[
            pl.BlockSpec(memory_space=pl.ANY),
            pl.BlockSpec((3, BN), lambda i: (0, i)),
        ],
        out_specs=pl.BlockSpec((8, BN), lambda i: (16, i)),
        out_shape=jax.ShapeDtypeStruct((DIM + 3, N), jnp.float32),
        input_output_aliases={0: 0},
    )(feats_out, cent)


def kernel(ins_fea, embeddings, unq_inv, W_reduce):
    idx32 = unq_inv.astype(jnp.int32)
    idx_pad = jnp.pad(idx32, (0, N2 - N)).reshape(NW, CH, B)
    idx4 = jnp.concatenate([idx_pad + comp * M for comp in range(4)], axis=1)

    pad1 = lambda v: jnp.pad(v, (0, N2 - N))
    embT = jnp.concatenate([
        pad1(embeddings[:, 0]), pad1(embeddings[:, 1]), pad1(embeddings[:, 2]),
        pad1(jnp.ones((N,), jnp.float32))])
    zeros4 = jnp.zeros((4 * M,), jnp.float32)

    parts = _segsum_call(embT, idx4, zeros4)
    table = _shift_call(parts)
    cent = _gather_call(table, idx4).reshape(3, N2)
    feats_t = _mm_call(ins_fea, W_reduce.T)
    out_t = _cw_call(feats_t, cent)
    return out_t.T


# speedup vs baseline: 1.0447x; 1.0447x over previous
import jax
import jax.numpy as jnp
from jax import lax
from jax.experimental import pallas as pl
from jax.experimental.pallas import tpu as pltpu
from jax.experimental.pallas import tpu_sc as plsc

N = 100000
IN_CH = 384
DIM = 128
M = 4096
BW2 = 4.0
ITERS = 4

NC = 2
NS = 16
NW = NC * NS
B = 128
CH = 25
ROWS_W = B * CH
N2 = NW * ROWS_W

BI = 256
NBLK = M // BI


def _segsum_body(emb_hbm, idx_hbm, zeros_hbm, out_hbm, idx_v, emb_v, acc_sh,
                 sem):
    c = lax.axis_index("c")
    s = lax.axis_index("s")
    w = c * NS + s

    @pl.when(s == 0)
    def _():
        pltpu.sync_copy(zeros_hbm, acc_sh)

    for comp in range(4):
        pltpu.sync_copy(emb_hbm.at[pl.ds(comp * N2 + w * ROWS_W, ROWS_W)],
                        emb_v.at[pl.ds(comp * ROWS_W, ROWS_W)])
    pltpu.sync_copy(idx_hbm.at[w], idx_v)
    plsc.subcore_barrier()

    for g in range(4):
        descs = [
            pltpu.async_copy(emb_v.at[pl.ds((g * CH + k) * B, B)],
                             acc_sh.at[idx_v.at[g * CH + k]], sem, add=True)
            for k in range(CH)
        ]
        for d in descs:
            d.wait()
    plsc.subcore_barrier()

    @pl.when(s == 0)
    def _():
        pltpu.sync_copy(acc_sh, out_hbm.at[c])


_segsum_call = pl.kernel(
    _segsum_body,
    out_type=jax.ShapeDtypeStruct((NC, 4 * M), jnp.float32),
    mesh=plsc.VectorSubcoreMesh(core_axis_name="c", subcore_axis_name="s"),
    scratch_types=[
        pltpu.VMEM((4 * CH, B), jnp.int32),
        pltpu.VMEM((4 * ROWS_W,), jnp.float32),
        pltpu.VMEM_SHARED((4 * M,), jnp.float32),
        pltpu.SemaphoreType.DMA,
    ],
)


def _shift_body(parts_ref, out_ref, k_ref, col_ref, yt_ref, yt2_ref, yhl_ref):
    rows = []
    for comp in range(4):
        rows.append(parts_ref[0:1, pl.ds(comp * M, M)]
                    + parts_ref[1:2, pl.ds(comp * M, M)])
    cntr = jnp.maximum(rows[3], 1.0)
    xr = rows[0] / cntr
    yr = rows[1] / cntr
    zr = rows[2] / cntr
    sqr = xr * xr + yr * yr + zr * zr
    onesr = jnp.ones((1, M), jnp.float32)
    zerosr = jnp.zeros((4, M), jnp.float32)
    yt_ref[...] = jnp.concatenate([xr, yr, zr, onesr, zerosr], axis=0)
    col_ref[...] = jnp.transpose(
        jnp.concatenate([xr, yr, zr, sqr], axis=0))

    def kb(i, carry):
        r = pl.ds(i * BI, BI)
        xi = col_ref[r, 0:1]
        yi = col_ref[r, 1:2]
        zi = col_ref[r, 2:3]
        sqi = col_ref[r, 3:4]
        d2 = (sqi + sqr) - 2.0 * (xi * xr + yi * yr + zi * zr)
        k_ref[r, :] = (d2 <= BW2).astype(jnp.bfloat16)
        return carry

    lax.fori_loop(0, NBLK, kb, 0)

    def it(t, carry):
        yt = yt_ref[...]
        yth = yt.astype(jnp.bfloat16)
        ytl = (yt - yth.astype(jnp.float32)).astype(jnp.bfloat16)
        yhl_ref[...] = jnp.concatenate([yth, ytl], axis=0)

        def mm(i, c2):
            cslice = pl.ds(i * BI, BI)
            kcol = k_ref[:, cslice]
            z = jnp.dot(yhl_ref[...], kcol,
                        preferred_element_type=jnp.float32)
            zs = z[0:8, :] + z[8:16, :]
            yt2_ref[:, cslice] = zs / zs[3:4, :]
            return c2

        lax.fori_loop(0, NBLK, mm, 0)
        yt_ref[...] = yt2_ref[...]
        return carry

    lax.fori_loop(0, ITERS, it, 0)
    for comp in range(3):
        out_ref[pl.ds(comp * M, M)] = jnp.reshape(yt_ref[comp:comp + 1, :], (M,))
    out_ref[pl.ds(3 * M, M)] = jnp.reshape(yt_ref[3:4, :], (M,))


def _shift_call(parts2):
    return pl.pallas_call(
        _shift_body,
        out_shape=jax.ShapeDtypeStruct((4 * M,), jnp.float32),
        scratch_shapes=[
            pltpu.VMEM((M, M), jnp.bfloat16),
            pltpu.VMEM((M, 4), jnp.float32),
            pltpu.VMEM((8, M), jnp.float32),
            pltpu.VMEM((8, M), jnp.float32),
            pltpu.VMEM((16, M), jnp.bfloat16),
        ],
    )(parts2)


def _gather_body(tab_hbm, idx_hbm, out_hbm, tab_v, idx_v, out_v):
    c = lax.axis_index("c")
    s = lax.axis_index("s")
    w = c * NS + s
    pltpu.sync_copy(tab_hbm, tab_v)
    pltpu.sync_copy(idx_hbm.at[w], idx_v)

    def body(j, carry):
        def inner(k2, c2):
            idx16 = idx_v[j, pl.ds(k2 * 16, 16)]
            vals = plsc.load_gather(tab_v, [idx16])
            out_v[pl.ds(j * B + k2 * 16, 16)] = vals
            return c2

        lax.fori_loop(0, B // 16, inner, 0)
        return carry

    lax.fori_loop(0, 3 * CH, body, 0)
    for comp in range(3):
        pltpu.sync_copy(out_v.at[pl.ds(comp * ROWS_W, ROWS_W)],
                        out_hbm.at[pl.ds(comp * N2 + w * ROWS_W, ROWS_W)])


_gather_call = pl.kernel(
    _gather_body,
    out_type=jax.ShapeDtypeStruct((3 * N2,), jnp.float32),
    mesh=plsc.VectorSubcoreMesh(core_axis_name="c", subcore_axis_name="s"),
    compiler_params=pltpu.CompilerParams(needs_layout_passes=False),
    scratch_types=[
        pltpu.VMEM((4 * M,), jnp.float32),
        pltpu.VMEM((4 * CH, B), jnp.int32),
        pltpu.VMEM((3 * ROWS_W,), jnp.float32),
    ],
)


BN = 4096


def _mm_body(x_ref, wt_ref, c_ref, o_ref):
    ft = lax.dot_general(wt_ref[...], x_ref[...], (((1,), (1,)), ((), ())),
                         preferred_element_type=jnp.float32)
    o_ref[...] = jnp.concatenate([ft, c_ref[...]], axis=0)


def _mm_call(ins_fea, wt, cent):
    return pl.pallas_call(
        _mm_body,
        grid=((N + BN - 1) // BN,),
        in_specs=[
            pl.BlockSpec((BN, IN_CH), lambda i: (i, 0)),
            pl.BlockSpec((DIM, IN_CH), lambda i: (0, 0)),
            pl.BlockSpec((3, BN), lambda i: (0, i)),
        ],
        out_specs=pl.BlockSpec((DIM + 3, BN), lambda i: (0, i)),
        out_shape=jax.ShapeDtypeStruct((DIM + 3, N), jnp.float32),
    )(ins_fea, wt, cent)


def kernel(ins_fea, embeddings, unq_inv, W_reduce):
    idx32 = unq_inv.astype(jnp.int32)
    idx_pad = jnp.pad(idx32, (0, N2 - N)).reshape(NW, CH, B)
    idx4 = jnp.concatenate([idx_pad + comp * M for comp in range(4)], axis=1)

    pad1 = lambda v: jnp.pad(v, (0, N2 - N))
    embT = jnp.concatenate([
        pad1(embeddings[:, 0]), pad1(embeddings[:, 1]), pad1(embeddings[:, 2]),
        pad1(jnp.ones((N,), jnp.float32))])
    zeros4 = jnp.zeros((4 * M,), jnp.float32)

    parts = _segsum_call(embT, idx4, zeros4)
    table = _shift_call(parts)
    cent = _gather_call(table, idx4).reshape(3, N2)
    out_t = _mm_call(ins_fea, W_reduce.T, cent)
    return out_t.T


# speedup vs baseline: 1.1639x; 1.1141x over previous
import jax
import jax.numpy as jnp
from jax import lax
from jax.experimental import pallas as pl
from jax.experimental.pallas import tpu as pltpu
from jax.experimental.pallas import tpu_sc as plsc

N = 100000
IN_CH = 384
DIM = 128
M = 4096
BW2 = 4.0
ITERS = 4

NC = 2
NS = 16
NW = NC * NS
B = 128
CH = 25
ROWS_W = B * CH
N2 = NW * ROWS_W

BI = 256
NBLK = M // BI


def _segsum_body(emb_hbm, idx_hbm, zeros_hbm, out_hbm, idx_v, emb_v, acc_sh,
                 sem):
    c = lax.axis_index("c")
    s = lax.axis_index("s")
    w = c * NS + s

    @pl.when(s == 0)
    def _():
        pltpu.sync_copy(zeros_hbm, acc_sh)

    for comp in range(4):
        pltpu.sync_copy(emb_hbm.at[pl.ds(comp * N2 + w * ROWS_W, ROWS_W)],
                        emb_v.at[pl.ds(comp * ROWS_W, ROWS_W)])
    pltpu.sync_copy(idx_hbm.at[w], idx_v)
    plsc.subcore_barrier()

    for g in range(4):
        descs = [
            pltpu.async_copy(emb_v.at[pl.ds((g * CH + k) * B, B)],
                             acc_sh.at[idx_v.at[g * CH + k]], sem, add=True)
            for k in range(CH)
        ]
        for d in descs:
            d.wait()
    plsc.subcore_barrier()

    @pl.when(s == 0)
    def _():
        pltpu.sync_copy(acc_sh, out_hbm.at[c])


_segsum_call = pl.kernel(
    _segsum_body,
    out_type=jax.ShapeDtypeStruct((NC, 4 * M), jnp.float32),
    mesh=plsc.VectorSubcoreMesh(core_axis_name="c", subcore_axis_name="s"),
    scratch_types=[
        pltpu.VMEM((4 * CH, B), jnp.int32),
        pltpu.VMEM((4 * ROWS_W,), jnp.float32),
        pltpu.VMEM_SHARED((4 * M,), jnp.float32),
        pltpu.SemaphoreType.DMA,
    ],
)


def _shift_body(parts_ref, out_ref, k_ref, col_ref, yt_ref, yt2_ref, yhl_ref,
                tb_ref):
    rows = []
    for comp in range(4):
        rows.append(parts_ref[0:1, pl.ds(comp * M, M)]
                    + parts_ref[1:2, pl.ds(comp * M, M)])
    cntr = jnp.maximum(rows[3], 1.0)
    xr = rows[0] / cntr
    yr = rows[1] / cntr
    zr = rows[2] / cntr
    sqr = xr * xr + yr * yr + zr * zr
    onesr = jnp.ones((1, M), jnp.float32)
    zerosr = jnp.zeros((4, M), jnp.float32)
    yt_ref[...] = jnp.concatenate([xr, yr, zr, onesr, zerosr], axis=0)
    col_ref[...] = jnp.transpose(
        jnp.concatenate([xr, yr, zr, sqr], axis=0))
    tb_ref[...] = jnp.concatenate(
        [xr, yr, zr, jnp.zeros((1, M), jnp.float32)],
        axis=0).astype(jnp.bfloat16)

    def kb(i, carry):
        r = pl.ds(i * BI, BI)
        cb = col_ref[r, :].astype(jnp.bfloat16)
        p = jnp.dot(cb, tb_ref[...],
                    preferred_element_type=jnp.float32)
        sqi = col_ref[r, 3:4]
        d2 = (sqi + sqr) - 2.0 * p
        k_ref[r, :] = (d2 <= BW2).astype(jnp.bfloat16)
        return carry

    lax.fori_loop(0, NBLK, kb, 0)

    def it(t, carry):
        yhl_ref[...] = yt_ref[...].astype(jnp.bfloat16)

        def mm(i, c2):
            cslice = pl.ds(i * BI, BI)
            kcol = k_ref[:, cslice]
            zs = jnp.dot(yhl_ref[...], kcol,
                         preferred_element_type=jnp.float32)
            yt2_ref[:, cslice] = zs / zs[3:4, :]
            return c2

        lax.fori_loop(0, NBLK, mm, 0)
        yt_ref[...] = yt2_ref[...]
        return carry

    lax.fori_loop(0, ITERS, it, 0)
    for comp in range(3):
        out_ref[pl.ds(comp * M, M)] = jnp.reshape(yt_ref[comp:comp + 1, :], (M,))
    out_ref[pl.ds(3 * M, M)] = jnp.reshape(yt_ref[3:4, :], (M,))


def _shift_call(parts2):
    return pl.pallas_call(
        _shift_body,
        out_shape=jax.ShapeDtypeStruct((4 * M,), jnp.float32),
        scratch_shapes=[
            pltpu.VMEM((M, M), jnp.bfloat16),
            pltpu.VMEM((M, 4), jnp.float32),
            pltpu.VMEM((8, M), jnp.float32),
            pltpu.VMEM((8, M), jnp.float32),
            pltpu.VMEM((8, M), jnp.bfloat16),
            pltpu.VMEM((4, M), jnp.bfloat16),
        ],
    )(parts2)


def _gather_body(tab_hbm, idx_hbm, out_hbm, tab_v, idx_v, out_v):
    c = lax.axis_index("c")
    s = lax.axis_index("s")
    w = c * NS + s
    pltpu.sync_copy(tab_hbm, tab_v)
    pltpu.sync_copy(idx_hbm.at[w], idx_v)

    def body(j, carry):
        def inner(k2, c2):
            idx16 = idx_v[j, pl.ds(k2 * 16, 16)]
            vals = plsc.load_gather(tab_v, [idx16])
            out_v[pl.ds(j * B + k2 * 16, 16)] = vals
            return c2

        lax.fori_loop(0, B // 16, inner, 0)
        return carry

    lax.fori_loop(0, 3 * CH, body, 0)
    for comp in range(3):
        pltpu.sync_copy(out_v.at[pl.ds(comp * ROWS_W, ROWS_W)],
                        out_hbm.at[pl.ds(comp * N2 + w * ROWS_W, ROWS_W)])


_gather_call = pl.kernel(
    _gather_body,
    out_type=jax.ShapeDtypeStruct((3 * N2,), jnp.float32),
    mesh=plsc.VectorSubcoreMesh(core_axis_name="c", subcore_axis_name="s"),
    compiler_params=pltpu.CompilerParams(needs_layout_passes=False),
    scratch_types=[
        pltpu.VMEM((4 * M,), jnp.float32),
        pltpu.VMEM((4 * CH, B), jnp.int32),
        pltpu.VMEM((3 * ROWS_W,), jnp.float32),
    ],
)


BN = 4096


def _mm_body(x_ref, wt_ref, c_ref, o_ref):
    ft = lax.dot_general(wt_ref[...].astype(jnp.bfloat16),
                         x_ref[...].astype(jnp.bfloat16),
                         (((1,), (1,)), ((), ())),
                         preferred_element_type=jnp.float32)
    o_ref[...] = jnp.concatenate([ft, c_ref[...]], axis=0)


def _mm_call(ins_fea, wt, cent):
    return pl.pallas_call(
        _mm_body,
        grid=((N + BN - 1) // BN,),
        in_specs=[
            pl.BlockSpec((BN, IN_CH), lambda i: (i, 0)),
            pl.BlockSpec((DIM, IN_CH), lambda i: (0, 0)),
            pl.BlockSpec((3, BN), lambda i: (0, i)),
        ],
        out_specs=pl.BlockSpec((DIM + 3, BN), lambda i: (0, i)),
        out_shape=jax.ShapeDtypeStruct((DIM + 3, N), jnp.float32),
    )(ins_fea, wt, cent)


def kernel(ins_fea, embeddings, unq_inv, W_reduce):
    idx32 = unq_inv.astype(jnp.int32)
    idx_pad = jnp.pad(idx32, (0, N2 - N)).reshape(NW, CH, B)
    idx4 = jnp.concatenate([idx_pad + comp * M for comp in range(4)], axis=1)

    pad1 = lambda v: jnp.pad(v, (0, N2 - N))
    embT = jnp.concatenate([
        pad1(embeddings[:, 0]), pad1(embeddings[:, 1]), pad1(embeddings[:, 2]),
        pad1(jnp.ones((N,), jnp.float32))])
    zeros4 = jnp.zeros((4 * M,), jnp.float32)

    parts = _segsum_call(embT, idx4, zeros4)
    table = _shift_call(parts)
    cent = _gather_call(table, idx4).reshape(3, N2)
    out_t = _mm_call(ins_fea, W_reduce.T, cent)
    return out_t.T
